# baseline (speedup 1.0000x reference)
import jax
import jax.numpy as jnp
from jax.experimental import pallas as pl
from jax.experimental.pallas import tpu as pltpu

B, T, S, D = 32, 8, 2048, 1024
S2 = S // 2


def _fused_kernel(x_ref, mba_ref, mbb_ref, win_ref, wout_hbm,
                  scores_ref, alpha_ref, attn_ref,
                  h_scr, c_scr, wout_scr, wout_sem):
    b = pl.program_id(0)

    @pl.when(b == 0)
    def _prologue():
        pltpu.make_async_copy(wout_hbm, wout_scr, wout_sem).start()
        h_scr[...] = jax.lax.dot_general(
            x_ref[...], win_ref[...], (((1,), (1,)), ((), ())),
            preferred_element_type=jnp.float32)

    h = h_scr[pl.ds(b * T, T), :]
    mba = mba_ref[0]
    mbb = mbb_ref[0]
    sa = jax.lax.dot_general(h, mba, (((1,), (1,)), ((), ())),
                             preferred_element_type=jnp.float32)
    sb = jax.lax.dot_general(h, mbb, (((1,), (1,)), ((), ())),
                             preferred_element_type=jnp.float32)
    s = jnp.concatenate([sa, sb], axis=-1)
    scores_ref[0] = s
    m = jnp.max(s, axis=-1, keepdims=True)
    e = jnp.exp(s - m)
    denom = jnp.sum(e, axis=-1, keepdims=True)
    a = e / denom
    alpha_ref[0] = a
    c_scr[pl.ds(b * T, T), :] = (
        jnp.dot(a[:, :S2], mba, preferred_element_type=jnp.float32)
        + jnp.dot(a[:, S2:], mbb, preferred_element_type=jnp.float32))

    @pl.when(b == B - 1)
    def _epilogue():
        pltpu.make_async_copy(wout_hbm, wout_scr, wout_sem).wait()
        w_c = wout_scr[:, :D]
        w_x = wout_scr[:, D:]
        out = (jax.lax.dot_general(c_scr[...], w_c, (((1,), (1,)), ((), ())),
                                   preferred_element_type=jnp.float32)
               + jax.lax.dot_general(x_ref[...], w_x, (((1,), (1,)), ((), ())),
                                     preferred_element_type=jnp.float32))
        attn_ref[...] = jnp.tanh(out)


def kernel(input, memory_bank, W_in, W_out):
    x2d = input.reshape(B * T, D)

    scores, alpha, attn2d = pl.pallas_call(
        _fused_kernel,
        grid=(B,),
        in_specs=[
            pl.BlockSpec((B * T, D), lambda b: (0, 0)),
            pl.BlockSpec((1, S2, D), lambda b: (b, 0, 0)),
            pl.BlockSpec((1, S2, D), lambda b: (b, 1, 0)),
            pl.BlockSpec((D, D), lambda b: (0, 0)),
            pl.BlockSpec(memory_space=pl.ANY),
        ],
        out_specs=(
            pl.BlockSpec((1, T, S), lambda b: (b, 0, 0)),
            pl.BlockSpec((1, T, S), lambda b: (b, 0, 0)),
            pl.BlockSpec((B * T, D), lambda b: (0, 0)),
        ),
        out_shape=(
            jax.ShapeDtypeStruct((B, T, S), jnp.float32),
            jax.ShapeDtypeStruct((B, T, S), jnp.float32),
            jax.ShapeDtypeStruct((B * T, D), jnp.float32),
        ),
        scratch_shapes=[
            pltpu.VMEM((B * T, D), jnp.float32),
            pltpu.VMEM((B * T, D), jnp.float32),
            pltpu.VMEM((D, 2 * D), jnp.float32),
            pltpu.SemaphoreType.DMA,
        ],
    )(x2d, memory_bank, memory_bank, W_in, W_out)

    return (attn2d.reshape(B, T, D), alpha, scores)


# speedup vs baseline: 1.0038x; 1.0038x over previous
import jax
import jax.numpy as jnp
from jax.experimental import pallas as pl
from jax.experimental.pallas import tpu as pltpu

B, T, S, D = 32, 8, 2048, 1024


def _fused_kernel(x_ref, mb_ref, win_ref, wout_hbm,
                  scores_ref, alpha_ref, attn_ref,
                  h_scr, c_scr, wout_scr, wout_sem):
    b = pl.program_id(0)

    @pl.when(b == 0)
    def _prologue():
        pltpu.make_async_copy(wout_hbm, wout_scr, wout_sem).start()
        h_scr[...] = jax.lax.dot_general(
            x_ref[...], win_ref[...], (((1,), (1,)), ((), ())),
            preferred_element_type=jnp.float32)

    h = h_scr[pl.ds(b * T, T), :]
    mb = mb_ref[0]
    s = jax.lax.dot_general(h, mb, (((1,), (1,)), ((), ())),
                            preferred_element_type=jnp.float32)
    scores_ref[0] = s
    m = jnp.max(s, axis=-1, keepdims=True)
    e = jnp.exp(s - m)
    denom = jnp.sum(e, axis=-1, keepdims=True)
    a = e / denom
    alpha_ref[0] = a
    c_scr[pl.ds(b * T, T), :] = jnp.dot(
        a.astype(jnp.bfloat16), mb.astype(jnp.bfloat16),
        preferred_element_type=jnp.float32)

    @pl.when(b == B - 1)
    def _epilogue():
        pltpu.make_async_copy(wout_hbm, wout_scr, wout_sem).wait()
        w_c = wout_scr[:, :D]
        w_x = wout_scr[:, D:]
        out = (jax.lax.dot_general(c_scr[...], w_c, (((1,), (1,)), ((), ())),
                                   preferred_element_type=jnp.float32)
               + jax.lax.dot_general(x_ref[...], w_x, (((1,), (1,)), ((), ())),
                                     preferred_element_type=jnp.float32))
        attn_ref[...] = jnp.tanh(out)


def kernel(input, memory_bank, W_in, W_out):
    x2d = input.reshape(B * T, D)

    scores, alpha, attn2d = pl.pallas_call(
        _fused_kernel,
        grid=(B,),
        in_specs=[
            pl.BlockSpec((B * T, D), lambda b: (0, 0)),
            pl.BlockSpec((1, S, D), lambda b: (b, 0, 0)),
            pl.BlockSpec((D, D), lambda b: (0, 0)),
            pl.BlockSpec(memory_space=pl.ANY),
        ],
        out_specs=(
            pl.BlockSpec((1, T, S), lambda b: (b, 0, 0)),
            pl.BlockSpec((1, T, S), lambda b: (b, 0, 0)),
            pl.BlockSpec((B * T, D), lambda b: (0, 0)),
        ),
        out_shape=(
            jax.ShapeDtypeStruct((B, T, S), jnp.float32),
            jax.ShapeDtypeStruct((B, T, S), jnp.float32),
            jax.ShapeDtypeStruct((B * T, D), jnp.float32),
        ),
        scratch_shapes=[
            pltpu.VMEM((B * T, D), jnp.float32),
            pltpu.VMEM((B * T, D), jnp.float32),
            pltpu.VMEM((D, 2 * D), jnp.float32),
            pltpu.SemaphoreType.DMA,
        ],
    )(x2d, memory_bank, W_in, W_out)

    return (attn2d.reshape(B, T, D), alpha, scores)


# speedup vs baseline: 1.1117x; 1.1075x over previous
import jax
import jax.numpy as jnp
from jax.experimental import pallas as pl
from jax.experimental.pallas import tpu as pltpu

B, T, S, D = 32, 8, 2048, 1024


def _fused_kernel(x_ref, mb_hbm, win_ref, wout_hbm,
                  scores_ref, alpha_ref, attn_ref,
                  h_scr, c_scr, wout_scr, mb_buf, mb_sem, wout_sem):
    b = pl.program_id(0)

    def mb_copy(i, slot):
        return pltpu.make_async_copy(mb_hbm.at[i], mb_buf.at[slot],
                                     mb_sem.at[slot])

    slot = jax.lax.rem(b, 2)

    @pl.when(b == 0)
    def _prologue():
        mb_copy(0, 0).start()
        mb_copy(1, 1).start()
        pltpu.make_async_copy(wout_hbm, wout_scr, wout_sem).start()
        h_scr[...] = jax.lax.dot_general(
            x_ref[...], win_ref[...], (((1,), (1,)), ((), ())),
            preferred_element_type=jnp.float32)

    mb_copy(b, slot).wait()

    h = h_scr[pl.ds(b * T, T), :]
    mb = mb_buf[slot]
    s = jax.lax.dot_general(h, mb, (((1,), (1,)), ((), ())),
                            preferred_element_type=jnp.float32)
    scores_ref[0] = s
    m = jnp.max(s, axis=-1, keepdims=True)
    e = jnp.exp(s - m)
    denom = jnp.sum(e, axis=-1, keepdims=True)
    a = e / denom
    alpha_ref[0] = a
    c_scr[pl.ds(b * T, T), :] = jnp.dot(a, mb,
                                        preferred_element_type=jnp.float32)

    @pl.when(b < B - 2)
    def _prefetch_next():
        mb_copy(b + 2, slot).start()

    @pl.when(b == B - 1)
    def _epilogue():
        pltpu.make_async_copy(wout_hbm, wout_scr, wout_sem).wait()
        w_c = wout_scr[:, :D]
        w_x = wout_scr[:, D:]
        out = (jax.lax.dot_general(c_scr[...], w_c, (((1,), (1,)), ((), ())),
                                   preferred_element_type=jnp.float32)
               + jax.lax.dot_general(x_ref[...], w_x, (((1,), (1,)), ((), ())),
                                     preferred_element_type=jnp.float32))
        attn_ref[...] = jnp.tanh(out)


def kernel(input, memory_bank, W_in, W_out):
    x2d = input.reshape(B * T, D)

    scores, alpha, attn2d = pl.pallas_call(
        _fused_kernel,
        grid=(B,),
        in_specs=[
            pl.BlockSpec((B * T, D), lambda b: (0, 0)),
            pl.BlockSpec(memory_space=pl.ANY),
            pl.BlockSpec((D, D), lambda b: (0, 0)),
            pl.BlockSpec(memory_space=pl.ANY),
        ],
        out_specs=(
            pl.BlockSpec((1, T, S), lambda b: (b, 0, 0)),
            pl.BlockSpec((1, T, S), lambda b: (b, 0, 0)),
            pl.BlockSpec((B * T, D), lambda b: (0, 0)),
        ),
        out_shape=(
            jax.ShapeDtypeStruct((B, T, S), jnp.float32),
            jax.ShapeDtypeStruct((B, T, S), jnp.float32),
            jax.ShapeDtypeStruct((B * T, D), jnp.float32),
        ),
        scratch_shapes=[
            pltpu.VMEM((B * T, D), jnp.float32),
            pltpu.VMEM((B * T, D), jnp.float32),
            pltpu.VMEM((D, 2 * D), jnp.float32),
            pltpu.VMEM((2, S, D), jnp.float32),
            pltpu.SemaphoreType.DMA((2,)),
            pltpu.SemaphoreType.DMA,
        ],
    )(x2d, memory_bank, W_in, W_out)

    return (attn2d.reshape(B, T, D), alpha, scores)
